# baseline (speedup 1.0000x reference)
import functools

import jax
import jax.numpy as jnp
from jax import lax
from jax.experimental import pallas as pl
from jax.experimental.pallas import tpu as pltpu
from jax.experimental.pallas import tpu_sc as plsc

D_MODEL = 768
MAX_LEN = 4096
BATCH = 4
LANES = 16
NUM_CORES = 2
NUM_SUBCORES = 16
NUM_WORKERS = NUM_CORES * NUM_SUBCORES
ROWS_PER_WORKER = MAX_LEN // NUM_WORKERS
GROUPS = ROWS_PER_WORKER // LANES
SUB_ROWS = 32
NUM_SUB = ROWS_PER_WORKER // SUB_ROWS

SC_BATCH = 3
TC_BATCHES = 3
TC_BLK = 512


def _sc_body(sym_hbm, pe_hbm, out_hbm, pe_v, sym_v, zero_v, ssem, wsem,
             *rsems):
    wid = lax.axis_index("s") * NUM_CORES + lax.axis_index("c")
    base = wid * ROWS_PER_WORKER

    sym_copy = pltpu.make_async_copy(
        sym_hbm.at[SC_BATCH, pl.ds(base, ROWS_PER_WORKER)], sym_v, ssem)
    sym_copy.start()

    reads = []
    for k in range(NUM_SUB):
        c = pltpu.make_async_copy(
            pe_hbm.at[0, pl.ds(base + k * SUB_ROWS, SUB_ROWS)],
            pe_v.at[pl.ds(k * SUB_ROWS, SUB_ROWS)],
            rsems[k],
        )
        c.start()
        reads.append(c)

    writes = []
    for k in range(NUM_SUB):
        reads[k].wait()
        c = pltpu.make_async_copy(
            pe_v.at[pl.ds(k * SUB_ROWS, SUB_ROWS)],
            out_hbm.at[0, pl.ds(base + k * SUB_ROWS, SUB_ROWS)],
            wsem,
        )
        c.start()
        writes.append(c)

    zeros = jnp.zeros((LANES,), jnp.float32)
    for j in range(D_MODEL // LANES):
        zero_v[pl.ds(j * LANES, LANES)] = zeros

    sym_copy.wait()
    for c in writes:
        c.wait()

    lane_iota = lax.iota(jnp.int32, LANES)

    def group_body(g, _):
        sv = sym_v[pl.ds(g * LANES, LANES)]
        pad = sv == 0
        n_pad = plsc.all_reduce_population_count(pad)[0]

        @pl.when(n_pad > 0)
        def _():
            def lane_body(i, _):
                is_pad = plsc.all_reduce_population_count(
                    jnp.logical_and(pad, lane_iota == i))[0]

                @pl.when(is_pad > 0)
                def _():
                    row = base + g * LANES + i
                    pltpu.sync_copy(zero_v, out_hbm.at[0, row])

                return 0

            lax.fori_loop(0, LANES, lane_body, 0)

        return 0

    lax.fori_loop(0, GROUPS, group_body, 0)


@functools.partial(
    pl.kernel,
    out_type=jax.ShapeDtypeStruct((1, MAX_LEN, D_MODEL), jnp.float32),
    mesh=plsc.VectorSubcoreMesh(core_axis_name="c", subcore_axis_name="s"),
    compiler_params=pltpu.CompilerParams(needs_layout_passes=False),
    scratch_types=[
        pltpu.VMEM((ROWS_PER_WORKER, D_MODEL), jnp.float32),
        pltpu.VMEM((ROWS_PER_WORKER,), jnp.int32),
        pltpu.VMEM((D_MODEL,), jnp.float32),
        pltpu.SemaphoreType.DMA,
        pltpu.SemaphoreType.DMA,
    ] + [pltpu.SemaphoreType.DMA] * NUM_SUB,
)
def _sc_broadcast(sym_hbm, pe_hbm, out_hbm, pe_v, sym_v, zero_v, ssem, wsem,
                  *rsems):
    _sc_body(sym_hbm, pe_hbm, out_hbm, pe_v, sym_v, zero_v, ssem, wsem,
             *rsems)


def _tc_body(sym_ref, pe_ref, out_ref):
    m = (sym_ref[...] != 0).astype(jnp.float32)
    pe = pe_ref[...]
    out_ref[...] = pe[None, :, :] * m[:, :, None]


def _tc_broadcast(sym, pe2d):
    return pl.pallas_call(
        _tc_body,
        grid=(MAX_LEN // TC_BLK,),
        in_specs=[
            pl.BlockSpec((TC_BATCHES, TC_BLK), lambda i: (0, i)),
            pl.BlockSpec((TC_BLK, D_MODEL), lambda i: (i, 0)),
        ],
        out_specs=pl.BlockSpec(
            (TC_BATCHES, TC_BLK, D_MODEL), lambda i: (0, i, 0)),
        out_shape=jax.ShapeDtypeStruct(
            (TC_BATCHES, MAX_LEN, D_MODEL), jnp.float32),
        compiler_params=pltpu.CompilerParams(
            dimension_semantics=("arbitrary",),
        ),
    )(sym[:TC_BATCHES], pe2d)


def kernel(symbol, positional_encoding):
    sym = symbol.astype(jnp.int32)
    pe2d = positional_encoding.reshape(MAX_LEN, D_MODEL)
    out_sc = _sc_broadcast(sym, positional_encoding)
    out_tc = _tc_broadcast(sym, pe2d)
    return jnp.concatenate([out_tc, out_sc], axis=0)
